# baseline (speedup 1.0000x reference)
import functools
import math

import jax
import jax.numpy as jnp
from jax import lax
from jax.experimental import pallas as pl
from jax.experimental.pallas import tpu as pltpu
from jax.experimental.pallas import tpu_sc as plsc

_NC = 2
_NS = 16
_NW = _NC * _NS
_LANES = 16
_CHUNK_ROWS = 64


def _sc_body(scale, n_chunks, d, in_hbm, out_hbm,
             buf0, buf1, si0, si1, so0, so1):
    wid = lax.axis_index("s") * _NC + lax.axis_index("c")
    base = wid * (n_chunks * _CHUNK_ROWS)
    bufs = (buf0, buf1)
    isems = (si0, si1)
    osems = (so0, so1)
    vecs_per_row = d // _LANES
    assert vecs_per_row & (vecs_per_row - 1) == 0
    row_shift = vecs_per_row.bit_length() - 1
    vecs = _CHUNK_ROWS * vecs_per_row

    def src(c):
        return in_hbm.at[pl.ds(base + c * _CHUNK_ROWS, _CHUNK_ROWS), :]

    def dst(c):
        return out_hbm.at[pl.ds(base + c * _CHUNK_ROWS, _CHUNK_ROWS), :]

    in_h = {0: pltpu.async_copy(src(0), bufs[0], isems[0])}
    out_h = {}
    for c in range(n_chunks):
        b = c % 2
        if c + 1 < n_chunks:
            if c - 1 >= 0:
                out_h[c - 1].wait()
            in_h[c + 1] = pltpu.async_copy(
                src(c + 1), bufs[1 - b], isems[1 - b])
        in_h[c].wait()

        @plsc.parallel_loop(0, vecs, step=1, unroll=8)
        def _scale_one(i):
            r = lax.shift_right_logical(i, row_shift)
            col = pl.multiple_of(
                lax.shift_left(lax.bitwise_and(i, vecs_per_row - 1), 4),
                _LANES)
            sl = pl.ds(col, _LANES)
            bufs[b][r, sl] = bufs[b][r, sl] * scale

        out_h[c] = pltpu.async_copy(bufs[b], dst(c), osems[b])
    out_h[n_chunks - 2].wait()
    out_h[n_chunks - 1].wait()


def kernel(x, embed):
    s = x.shape[-2]
    d = embed.shape[-1]
    scale = 1.0 / math.sqrt(d)
    assert s % (_NW * _CHUNK_ROWS) == 0 and d % _LANES == 0
    n_chunks = s // (_NW * _CHUNK_ROWS)

    mesh = plsc.VectorSubcoreMesh(
        core_axis_name="c", subcore_axis_name="s",
        num_cores=_NC, num_subcores=_NS)
    run = pl.kernel(
        functools.partial(_sc_body, scale, n_chunks, d),
        out_type=jax.ShapeDtypeStruct((s, d), embed.dtype),
        mesh=mesh,
        scratch_types=[
            pltpu.VMEM((_CHUNK_ROWS, d), embed.dtype),
            pltpu.VMEM((_CHUNK_ROWS, d), embed.dtype),
            pltpu.SemaphoreType.DMA,
            pltpu.SemaphoreType.DMA,
            pltpu.SemaphoreType.DMA,
            pltpu.SemaphoreType.DMA,
        ],
    )
    return run(embed[:s])


# speedup vs baseline: 2.0654x; 2.0654x over previous
import math

import jax
import jax.numpy as jnp
from jax.experimental import pallas as pl


def _scale_copy(e_ref, o_ref):
    o_ref[...] = e_ref[...] * (1.0 / math.sqrt(e_ref.shape[-1]))


def kernel(x, embed):
    s = x.shape[-2]
    d = embed.shape[-1]
    block = 1024
    return pl.pallas_call(
        _scale_copy,
        grid=(s // block,),
        in_specs=[pl.BlockSpec((block, d), lambda i: (i, 0))],
        out_specs=pl.BlockSpec((block, d), lambda i: (i, 0)),
        out_shape=jax.ShapeDtypeStruct((s, d), embed.dtype),
    )(embed[:s])


# speedup vs baseline: 2.2055x; 1.0678x over previous
import math

import jax
import jax.numpy as jnp
from jax.experimental import pallas as pl


def _scale_copy(e_ref, o_ref):
    o_ref[...] = e_ref[...] * (1.0 / math.sqrt(e_ref.shape[-1]))


def kernel(x, embed):
    s = x.shape[-2]
    d = embed.shape[-1]
    block = 2048
    return pl.pallas_call(
        _scale_copy,
        grid=(s // block,),
        in_specs=[pl.BlockSpec((block, d), lambda i: (i, 0))],
        out_specs=pl.BlockSpec((block, d), lambda i: (i, 0)),
        out_shape=jax.ShapeDtypeStruct((s, d), embed.dtype),
    )(embed[:s])
